# baseline (speedup 1.0000x reference)
import functools

import jax
import jax.numpy as jnp
from jax import lax
from jax.experimental import pallas as pl
from jax.experimental.pallas import tpu as pltpu
from jax.experimental.pallas import tpu_sc as plsc

_N = 10000
_NP = 10240
_NH = 5120
_NA = 5248
_DR = 48
_F = 512
_NCHUNK = 4
_CW = 128
_EPT = 11264
_EB = 88
_EPAD = 16 * _EPT
_NS = 16


def _mm_kernel(x_ref, w_ref, o_ref):
    o_ref[...] = jnp.dot(x_ref[...], w_ref[...],
                         preferred_element_type=jnp.float32)


def _matmul(x, w, bn=2000):
    n, d = x.shape
    k = w.shape[1]
    return pl.pallas_call(
        _mm_kernel,
        grid=(n // bn,),
        in_specs=[
            pl.BlockSpec((bn, d), lambda i: (i, 0)),
            pl.BlockSpec((d, k), lambda i: (0, 0)),
        ],
        out_specs=pl.BlockSpec((bn, k), lambda i: (i, 0)),
        out_shape=jax.ShapeDtypeStruct((n, k), jnp.float32),
    )(x, w)


def _sc_edge_kernel(hflat, src, dst2d, asrc, adst, out,
                    src_v, dst_v, asrc_v, adst_v, dloc_v, p_v,
                    rows2_v, idxA_v, idxB_v, dcl2_v,
                    idx48_v, semA, semB,
                    acc_s, dtot_s):
    cid = lax.axis_index("c")
    sid = lax.axis_index("s")
    base_n = cid * _NH

    pltpu.sync_copy(src.at[pl.ds(sid * _EPT, _EPT)], src_v)
    pltpu.sync_copy(dst2d.at[pl.ds(sid * _EB, _EB), :], dst_v)

    zeros16 = jnp.zeros((16,), jnp.float32)
    iota16 = lax.iota(jnp.int32, 16)
    for g in range(3):
        idx48_v[pl.ds(g * 16, 16)] = iota16 + g * 16

    def _zero_rows2(r, _):
        for k in range(_CW // 16):
            rows2_v[r, pl.ds(k * 16, 16)] = zeros16
        return _

    for head in range(2):
        pltpu.sync_copy(asrc.at[head], asrc_v)
        pltpu.sync_copy(adst.at[head], adst_v)

        def _zero_dloc(r, _):
            for k in range(_CW // 16):
                dloc_v[r, pl.ds(k * 16, 16)] = zeros16
            return _
        lax.fori_loop(0, _DR, _zero_dloc, None)

        @pl.when(sid == 0)
        def _():
            pltpu.sync_copy(dloc_v, dtot_s)

        def _b1(j, _):
            for k in range(8):
                base = j * 128 + k * 16
                s = src_v[pl.ds(base, 16)]
                d = dst_v[j, pl.ds(k * 16, 16)]
                a = plsc.load_gather(asrc_v, [s])
                b = plsc.load_gather(adst_v, [d])
                e = a + b
                e = jnp.where(e > 0, e, 0.2 * e)
                p = jnp.exp(e)
                p_v[pl.ds(base, 16)] = p
                dd = d - base_n
                ok = (dd >= 0) & (dd < _NH)
                dd = jnp.where(ok, dd, _NH)
                plsc.addupdate_scatter(
                    dloc_v, [lax.shift_right_logical(dd, 7),
                             jnp.bitwise_and(dd, 127)], p)
            return _
        lax.fori_loop(0, _EB, _b1, None)

        plsc.subcore_barrier()
        pltpu.sync_copy(dloc_v, dtot_s.at[idx48_v], add=True)
        plsc.subcore_barrier()
        pltpu.sync_copy(dtot_s, dloc_v)

        def _b2(j, _):
            for k in range(8):
                base = j * 128 + k * 16
                d = dst_v[j, pl.ds(k * 16, 16)]
                dd = d - base_n
                ok = (dd >= 0) & (dd < _NH)
                dd = jnp.where(ok, dd, _NH)
                dn = plsc.load_gather(
                    dloc_v, [lax.shift_right_logical(dd, 7),
                             jnp.bitwise_and(dd, 127)])
                p_v[pl.ds(base, 16)] = p_v[pl.ds(base, 16)] / (dn + 1e-16)
            return _
        lax.fori_loop(0, _EB, _b2, None)

        def _build(idx_ref, dcl_off, ebase, chunk):
            for k in range(4):
                s = src_v[pl.ds(ebase + k * 16, 16)]
                idx_ref[pl.ds(k * 16, 16)] = s * _NCHUNK + chunk
            for k in range(4):
                d = dst_v[ebase // 128, pl.ds(ebase % 128 + k * 16, 16)]
                dd = d - base_n
                ok = (dd >= 0) & (dd < _NH)
                spill = _NH + jnp.bitwise_and(d, 127)
                dcl2_v[pl.ds(dcl_off + k * 16, 16)] = jnp.where(ok, dd, spill)

        def _scale(row_off, ebase):
            def _sg(g, _2):
                av = p_v[pl.ds(ebase + g * 16, 16)]
                for rr in range(16):
                    al = av[rr]
                    r = row_off + g * 16 + rr
                    for k in range(_CW // 16):
                        rows2_v[r, pl.ds(k * 16, 16)] = (
                            rows2_v[r, pl.ds(k * 16, 16)] * al)
                return _2
            lax.fori_loop(0, 4, _sg, None)

        rowsA = rows2_v.at[pl.ds(0, 64), :]
        rowsB = rows2_v.at[pl.ds(64, 64), :]

        for half in range(2):
            chunk = head * 2 + half
            lax.fori_loop(0, 128, _zero_rows2, None)
            for i in range(3):
                c = sid + i * _NS

                @pl.when(c < _NA // 128)
                def _():
                    pltpu.sync_copy(rows2_v, acc_s.at[pl.ds(c * 128, 128), :])
            plsc.subcore_barrier()

            _build(idxA_v, 0, 0, chunk)
            pltpu.async_copy(hflat.at[idxA_v], rowsA, semA)

            def _cblk(m, _):
                baseA = m * 128
                baseB = m * 128 + 64
                pltpu.make_async_copy(
                    hflat.at[idxA_v], rowsA, semA).wait()
                _build(idxB_v, 64, baseB, chunk)
                pltpu.async_copy(hflat.at[idxB_v], rowsB, semB)
                _scale(0, baseA)
                pltpu.make_async_copy(
                    hflat.at[idxB_v], rowsB, semB).wait()
                _scale(64, baseB)
                pltpu.sync_copy(rows2_v, acc_s.at[dcl2_v], add=True)

                @pl.when(m < _EB - 1)
                def _():
                    _build(idxA_v, 0, baseB + 64, chunk)
                    pltpu.async_copy(hflat.at[idxA_v], rowsA, semA)
                return _
            lax.fori_loop(0, _EB, _cblk, None)
            plsc.subcore_barrier()
            pltpu.sync_copy(
                acc_s.at[pl.ds(sid * (_NH // _NS), _NH // _NS), :],
                out.at[pl.ds(base_n + sid * (_NH // _NS), _NH // _NS),
                       chunk, :])
            plsc.subcore_barrier()


def _sc_edge_phase(hflat, src, dst2d, asrc, adst):
    mesh = plsc.VectorSubcoreMesh(core_axis_name="c", subcore_axis_name="s")
    return pl.kernel(
        _sc_edge_kernel,
        out_type=jax.ShapeDtypeStruct((_NP, _NCHUNK, _CW), jnp.float32),
        mesh=mesh,
        compiler_params=pltpu.CompilerParams(needs_layout_passes=False),
        scratch_types=[
            pltpu.VMEM((_EPT,), jnp.int32),
            pltpu.VMEM((_EB, 128), jnp.int32),
            pltpu.VMEM((_NP,), jnp.float32),
            pltpu.VMEM((_NP,), jnp.float32),
            pltpu.VMEM((_DR, 128), jnp.float32),
            pltpu.VMEM((_EPT,), jnp.float32),
            pltpu.VMEM((128, _CW), jnp.float32),
            pltpu.VMEM((64,), jnp.int32),
            pltpu.VMEM((64,), jnp.int32),
            pltpu.VMEM((128,), jnp.int32),
            pltpu.VMEM((_DR,), jnp.int32),
            pltpu.SemaphoreType.DMA,
            pltpu.SemaphoreType.DMA,
            pltpu.VMEM_SHARED((_NA, _CW), jnp.float32),
            pltpu.VMEM_SHARED((_DR, 128), jnp.float32),
        ],
    )(hflat, src, dst2d, asrc, adst)


def _att_mats(att_src, att_dst, heads, dim):
    f = heads * dim
    m = jnp.zeros((f, 128), jnp.float32)
    for h in range(heads):
        m = m.at[h * dim:(h + 1) * dim, h].set(att_src[h])
        m = m.at[h * dim:(h + 1) * dim, heads + h].set(att_dst[h])
    return m


def _gat_layer(x, src, dst2d, W, att_src, att_dst, bias, heads, out_dim):
    h = _matmul(x, W)
    acat = _matmul(h, _att_mats(att_src, att_dst, heads, out_dim))
    a_src = acat[:, :heads]
    a_dst = acat[:, heads:2 * heads]
    pad = ((0, _NP - _N), (0, 0))
    asrcT = jnp.pad(a_src, pad).T
    adstT = jnp.pad(a_dst, pad).T
    hflat = h.reshape(_N * _NCHUNK, _CW)
    out = _sc_edge_phase(hflat, src, dst2d, asrcT, adstT)
    z = out.reshape(_NP, _F)[:_N]
    return z + bias


def kernel(x, edge_index, W1, att_src1, att_dst1, b1, W2, att_src2,
           att_dst2, b2):
    n = x.shape[0]
    loop = jnp.arange(n, dtype=jnp.int32)
    src = jnp.concatenate([edge_index[0].astype(jnp.int32), loop])
    dst = jnp.concatenate([edge_index[1].astype(jnp.int32), loop])
    npad = _EPAD - src.shape[0]
    src = jnp.concatenate([src, jnp.zeros((npad,), jnp.int32)])
    dst = jnp.concatenate([dst, jnp.full((npad,), _N, jnp.int32)])
    dst2d = dst.reshape(_EPAD // 128, 128)
    z = _gat_layer(x, src, dst2d, W1, att_src1, att_dst1, b1, 2, 256)
    z = jax.nn.relu(z)
    z = _gat_layer(z, src, dst2d, W2, att_src2, att_dst2, b2, 2, 256)
    return z


# speedup vs baseline: 1.0280x; 1.0280x over previous
import functools

import jax
import jax.numpy as jnp
from jax import lax
from jax.experimental import pallas as pl
from jax.experimental.pallas import tpu as pltpu
from jax.experimental.pallas import tpu_sc as plsc

_N = 10000
_NP = 10240
_NH = 5120
_NA = 5248
_DR = 48
_F = 512
_NCHUNK = 4
_CW = 128
_EPT = 11264
_EB = 88
_EPAD = 16 * _EPT
_NS = 16


def _mm_kernel(x_ref, w_ref, o_ref):
    o_ref[...] = jnp.dot(x_ref[...], w_ref[...],
                         preferred_element_type=jnp.float32)


def _matmul(x, w, bn=2000):
    n, d = x.shape
    k = w.shape[1]
    return pl.pallas_call(
        _mm_kernel,
        grid=(n // bn,),
        in_specs=[
            pl.BlockSpec((bn, d), lambda i: (i, 0)),
            pl.BlockSpec((d, k), lambda i: (0, 0)),
        ],
        out_specs=pl.BlockSpec((bn, k), lambda i: (i, 0)),
        out_shape=jax.ShapeDtypeStruct((n, k), jnp.float32),
    )(x, w)


def _sc_edge_kernel(hflat, src, dst2d, asrc, adst, out,
                    src_v, dst_v, asrc_v, adst_v, dloc_v, p_v,
                    rowsA_v, rowsB_v, idxA_v, idxB_v, dclA_v, dclB_v,
                    idx48_v, semA, semB,
                    acc_s, dtot_s):
    cid = lax.axis_index("c")
    sid = lax.axis_index("s")
    base_n = cid * _NH

    pltpu.sync_copy(src.at[pl.ds(sid * _EPT, _EPT)], src_v)
    pltpu.sync_copy(dst2d.at[pl.ds(sid * _EB, _EB), :], dst_v)

    zeros16 = jnp.zeros((16,), jnp.float32)
    iota16 = lax.iota(jnp.int32, 16)
    for g in range(3):
        idx48_v[pl.ds(g * 16, 16)] = iota16 + g * 16

    def _zero_rowsA(r, _):
        for k in range(_CW // 16):
            rowsA_v[r, pl.ds(k * 16, 16)] = zeros16
        return _

    for head in range(2):
        pltpu.sync_copy(asrc.at[head], asrc_v)
        pltpu.sync_copy(adst.at[head], adst_v)

        def _zero_dloc(r, _):
            for k in range(_CW // 16):
                dloc_v[r, pl.ds(k * 16, 16)] = zeros16
            return _
        lax.fori_loop(0, _DR, _zero_dloc, None)

        @pl.when(sid == 0)
        def _():
            pltpu.sync_copy(dloc_v, dtot_s)

        def _b1(j, _):
            for k in range(8):
                base = j * 128 + k * 16
                s = src_v[pl.ds(base, 16)]
                d = dst_v[j, pl.ds(k * 16, 16)]
                a = plsc.load_gather(asrc_v, [s])
                b = plsc.load_gather(adst_v, [d])
                e = a + b
                e = jnp.where(e > 0, e, 0.2 * e)
                p = jnp.exp(e)
                p_v[pl.ds(base, 16)] = p
                dd = d - base_n
                ok = (dd >= 0) & (dd < _NH)
                dd = jnp.where(ok, dd, _NH)
                plsc.addupdate_scatter(
                    dloc_v, [lax.shift_right_logical(dd, 7),
                             jnp.bitwise_and(dd, 127)], p)
            return _
        lax.fori_loop(0, _EB, _b1, None)

        plsc.subcore_barrier()
        pltpu.sync_copy(dloc_v, dtot_s.at[idx48_v], add=True)
        plsc.subcore_barrier()
        pltpu.sync_copy(dtot_s, dloc_v)

        def _b2(j, _):
            for k in range(8):
                base = j * 128 + k * 16
                d = dst_v[j, pl.ds(k * 16, 16)]
                dd = d - base_n
                ok = (dd >= 0) & (dd < _NH)
                dd = jnp.where(ok, dd, _NH)
                dn = plsc.load_gather(
                    dloc_v, [lax.shift_right_logical(dd, 7),
                             jnp.bitwise_and(dd, 127)])
                p_v[pl.ds(base, 16)] = p_v[pl.ds(base, 16)] / (dn + 1e-16)
            return _
        lax.fori_loop(0, _EB, _b2, None)

        def _build(idx_ref, dcl_ref, ebase, chunk):
            for k in range(4):
                s = src_v[pl.ds(ebase + k * 16, 16)]
                idx_ref[pl.ds(k * 16, 16)] = s * _NCHUNK + chunk
            for k in range(4):
                d = dst_v[ebase // 128, pl.ds(ebase % 128 + k * 16, 16)]
                dd = d - base_n
                ok = (dd >= 0) & (dd < _NH)
                spill = _NH + jnp.bitwise_and(d, 127)
                dcl_ref[pl.ds(k * 16, 16)] = jnp.where(ok, dd, spill)

        def _scale(rows_ref, ebase):
            def _sg(g, _2):
                av = p_v[pl.ds(ebase + g * 16, 16)]
                for rr in range(16):
                    al = av[rr]
                    r = g * 16 + rr
                    for k in range(_CW // 16):
                        rows_ref[r, pl.ds(k * 16, 16)] = (
                            rows_ref[r, pl.ds(k * 16, 16)] * al)
                return _2
            lax.fori_loop(0, 4, _sg, None)

        for half in range(2):
            chunk = head * 2 + half
            lax.fori_loop(0, 64, _zero_rowsA, None)
            for i in range(6):
                c = sid + i * _NS

                @pl.when(c < _NA // 64)
                def _():
                    pltpu.sync_copy(rowsA_v, acc_s.at[pl.ds(c * 64, 64), :])
            plsc.subcore_barrier()

            _build(idxA_v, dclA_v, 0, chunk)
            pltpu.async_copy(hflat.at[idxA_v], rowsA_v, semA)

            def _cblk(m, _):
                baseA = m * 128
                baseB = m * 128 + 64
                pltpu.make_async_copy(
                    hflat.at[idxA_v], rowsA_v, semA).wait()
                _build(idxB_v, dclB_v, baseB, chunk)
                pltpu.async_copy(hflat.at[idxB_v], rowsB_v, semB)
                _scale(rowsA_v, baseA)
                pltpu.sync_copy(rowsA_v, acc_s.at[dclA_v], add=True)
                pltpu.make_async_copy(
                    hflat.at[idxB_v], rowsB_v, semB).wait()

                @pl.when(m < _EB - 1)
                def _():
                    _build(idxA_v, dclA_v, baseB + 64, chunk)
                    pltpu.async_copy(hflat.at[idxA_v], rowsA_v, semA)
                _scale(rowsB_v, baseB)
                pltpu.sync_copy(rowsB_v, acc_s.at[dclB_v], add=True)
                return _
            lax.fori_loop(0, _EB, _cblk, None)
            plsc.subcore_barrier()
            pltpu.sync_copy(
                acc_s.at[pl.ds(sid * (_NH // _NS), _NH // _NS), :],
                out.at[pl.ds(base_n + sid * (_NH // _NS), _NH // _NS),
                       chunk, :])
            plsc.subcore_barrier()


def _sc_edge_phase(hflat, src, dst2d, asrc, adst):
    mesh = plsc.VectorSubcoreMesh(core_axis_name="c", subcore_axis_name="s")
    return pl.kernel(
        _sc_edge_kernel,
        out_type=jax.ShapeDtypeStruct((_NP, _NCHUNK, _CW), jnp.float32),
        mesh=mesh,
        compiler_params=pltpu.CompilerParams(needs_layout_passes=False),
        scratch_types=[
            pltpu.VMEM((_EPT,), jnp.int32),
            pltpu.VMEM((_EB, 128), jnp.int32),
            pltpu.VMEM((_NP,), jnp.float32),
            pltpu.VMEM((_NP,), jnp.float32),
            pltpu.VMEM((_DR, 128), jnp.float32),
            pltpu.VMEM((_EPT,), jnp.float32),
            pltpu.VMEM((64, _CW), jnp.float32),
            pltpu.VMEM((64, _CW), jnp.float32),
            pltpu.VMEM((64,), jnp.int32),
            pltpu.VMEM((64,), jnp.int32),
            pltpu.VMEM((64,), jnp.int32),
            pltpu.VMEM((64,), jnp.int32),
            pltpu.VMEM((_DR,), jnp.int32),
            pltpu.SemaphoreType.DMA,
            pltpu.SemaphoreType.DMA,
            pltpu.VMEM_SHARED((_NA, _CW), jnp.float32),
            pltpu.VMEM_SHARED((_DR, 128), jnp.float32),
        ],
    )(hflat, src, dst2d, asrc, adst)


def _att_mats(att_src, att_dst, heads, dim):
    f = heads * dim
    m = jnp.zeros((f, 128), jnp.float32)
    for h in range(heads):
        m = m.at[h * dim:(h + 1) * dim, h].set(att_src[h])
        m = m.at[h * dim:(h + 1) * dim, heads + h].set(att_dst[h])
    return m


def _gat_layer(x, src, dst2d, W, att_src, att_dst, bias, heads, out_dim):
    h = _matmul(x, W)
    acat = _matmul(h, _att_mats(att_src, att_dst, heads, out_dim))
    a_src = acat[:, :heads]
    a_dst = acat[:, heads:2 * heads]
    pad = ((0, _NP - _N), (0, 0))
    asrcT = jnp.pad(a_src, pad).T
    adstT = jnp.pad(a_dst, pad).T
    hflat = h.reshape(_N * _NCHUNK, _CW)
    out = _sc_edge_phase(hflat, src, dst2d, asrcT, adstT)
    z = out.reshape(_NP, _F)[:_N]
    return z + bias


def kernel(x, edge_index, W1, att_src1, att_dst1, b1, W2, att_src2,
           att_dst2, b2):
    n = x.shape[0]
    loop = jnp.arange(n, dtype=jnp.int32)
    src = jnp.concatenate([edge_index[0].astype(jnp.int32), loop])
    dst = jnp.concatenate([edge_index[1].astype(jnp.int32), loop])
    npad = _EPAD - src.shape[0]
    src = jnp.concatenate([src, jnp.zeros((npad,), jnp.int32)])
    dst = jnp.concatenate([dst, jnp.full((npad,), _N, jnp.int32)])
    dst2d = dst.reshape(_EPAD // 128, 128)
    z = _gat_layer(x, src, dst2d, W1, att_src1, att_dst1, b1, 2, 256)
    z = jax.nn.relu(z)
    z = _gat_layer(z, src, dst2d, W2, att_src2, att_dst2, b2, 2, 256)
    return z
